# baseline (speedup 1.0000x reference)
import functools

import jax
import jax.numpy as jnp
from jax.experimental import pallas as pl
from jax.experimental.pallas import tpu as pltpu

_NUM_CLASSES = 100000
_FEAT_LEN = 64
_BATCH = 1024
_BN = 2048


def _matmul_kernel(feats_ref, pid_ref, lut_ref, score_ref, labels_ref):
    f = feats_ref[...].astype(jnp.bfloat16)
    w = lut_ref[...].astype(jnp.bfloat16)
    score_ref[...] = jax.lax.dot_general(
        f, w, (((1,), (1,)), ((), ())), preferred_element_type=jnp.float32
    )
    p = pid_ref[...]
    labels_ref[...] = jnp.where((p < 0) | (p >= _NUM_CLASSES), -1, p)


@functools.partial(jax.jit, static_argnames=())
def kernel(feats, pid_labels, lookup_table):
    pid2d = pid_labels.reshape(8, 128)
    grid = (pl.cdiv(_NUM_CLASSES, _BN),)
    score, labels2d = pl.pallas_call(
        _matmul_kernel,
        grid=grid,
        in_specs=[
            pl.BlockSpec((_BATCH, _FEAT_LEN), lambda i: (0, 0)),
            pl.BlockSpec((8, 128), lambda i: (0, 0)),
            pl.BlockSpec((_BN, _FEAT_LEN), lambda i: (i, 0)),
        ],
        out_specs=[
            pl.BlockSpec((_BATCH, _BN), lambda i: (0, i)),
            pl.BlockSpec((8, 128), lambda i: (0, 0)),
        ],
        out_shape=[
            jax.ShapeDtypeStruct((_BATCH, _NUM_CLASSES), jnp.float32),
            jax.ShapeDtypeStruct((8, 128), jnp.int32),
        ],
        compiler_params=pltpu.CompilerParams(
            dimension_semantics=("arbitrary",),
        ),
    )(feats, pid2d, lookup_table)
    return (score, labels2d.reshape(-1))


# speedup vs baseline: 1.0109x; 1.0109x over previous
import functools

import jax
import jax.numpy as jnp
from jax.experimental import pallas as pl
from jax.experimental.pallas import tpu as pltpu

_NUM_CLASSES = 100000
_FEAT_LEN = 64
_BATCH = 1024
_BN = 4096


def _matmul_kernel(feats_ref, pid_ref, lut_ref, score_ref, labels_ref):
    f = feats_ref[...].astype(jnp.bfloat16)
    w = lut_ref[...].astype(jnp.bfloat16)
    score_ref[...] = jax.lax.dot_general(
        f, w, (((1,), (1,)), ((), ())), preferred_element_type=jnp.float32
    )
    p = pid_ref[...]
    labels_ref[...] = jnp.where((p < 0) | (p >= _NUM_CLASSES), -1, p)


@functools.partial(jax.jit, static_argnames=())
def kernel(feats, pid_labels, lookup_table):
    pid2d = pid_labels.reshape(8, 128)
    grid = (pl.cdiv(_NUM_CLASSES, _BN),)
    score, labels2d = pl.pallas_call(
        _matmul_kernel,
        grid=grid,
        in_specs=[
            pl.BlockSpec((_BATCH, _FEAT_LEN), lambda i: (0, 0)),
            pl.BlockSpec((8, 128), lambda i: (0, 0)),
            pl.BlockSpec((_BN, _FEAT_LEN), lambda i: (i, 0)),
        ],
        out_specs=[
            pl.BlockSpec((_BATCH, _BN), lambda i: (0, i)),
            pl.BlockSpec((8, 128), lambda i: (0, 0)),
        ],
        out_shape=[
            jax.ShapeDtypeStruct((_BATCH, _NUM_CLASSES), jnp.float32),
            jax.ShapeDtypeStruct((8, 128), jnp.int32),
        ],
        compiler_params=pltpu.CompilerParams(
            dimension_semantics=("arbitrary",),
        ),
    )(feats, pid2d, lookup_table)
    return (score, labels2d.reshape(-1))
